# Initial kernel scaffold; baseline (speedup 1.0000x reference)
import functools

import jax
import jax.numpy as jnp
from jax import lax
from jax.experimental import pallas as pl
from jax.experimental.pallas import tpu as pltpu
from jax.experimental.pallas import tpu_sc as plsc

_TEMP = 0.07
_NHARD = 16
_ROWS_PER_BLOCK = 256
_EPS = 1e-12


def _tc_body(feat_ref, pe_ref, cbt_ref, out_ref):
    f = feat_ref[...]
    pe = pe_ref[...]
    cbt = cbt_ref[...]

    b2 = jnp.sum(cbt * cbt, axis=0, keepdims=True)
    inv_bn = 1.0 / jnp.maximum(jnp.sqrt(b2), _EPS)
    qn = jnp.sqrt(jnp.sum(f * f, axis=1, keepdims=True))
    q = f / jnp.maximum(qn, _EPS)
    a2 = jnp.sum(pe * pe, axis=1, keepdims=True)

    p = jnp.dot(pe, cbt, preferred_element_type=jnp.float32)
    s = jnp.dot(q, cbt, preferred_element_type=jnp.float32) * inv_bn
    dist = jnp.sqrt(jnp.maximum(a2 + b2 - 2.0 * p, 0.0))

    work = dist
    tmin = None
    tk = None
    for i in range(_NHARD + 1):
        m = jnp.min(work, axis=1, keepdims=True)
        if i == 0:
            tmin = m
        tk = m
        if i < _NHARD:
            work = jnp.where(work <= m, jnp.float32(jnp.inf), work)

    neg_mask = jnp.logical_and(dist <= tk, dist > tmin)
    w = jnp.where(neg_mask, jnp.exp(s * (1.0 / _TEMP)), 0.0)
    negsum = jnp.sum(w, axis=1, keepdims=True)

    kpos = pe / jnp.maximum(jnp.sqrt(a2), _EPS)
    pos_l = jnp.sum(q * kpos, axis=1, keepdims=True) * (1.0 / _TEMP)
    row_loss = jnp.log(jnp.exp(pos_l) + negsum) - pos_l
    out_ref[...] = jnp.full((1, 1, 128), jnp.sum(row_loss), jnp.float32)


def _tc_loss(feat2d, pe, cbt):
    bl, c = feat2d.shape
    k = cbt.shape[1]
    r = _ROWS_PER_BLOCK
    nb = bl // r
    partial = pl.pallas_call(
        _tc_body,
        grid=(nb,),
        in_specs=[
            pl.BlockSpec((r, c), lambda i: (i, 0)),
            pl.BlockSpec((r, c), lambda i: (i, 0)),
            pl.BlockSpec((c, k), lambda i: (0, 0)),
        ],
        out_specs=pl.BlockSpec((1, 1, 128), lambda i: (i, 0, 0)),
        out_shape=jax.ShapeDtypeStruct((nb, 1, 128), jnp.float32),
    )(feat2d, pe, cbt)
    return jnp.sum(partial[:, 0, 0]) / bl


def _sc_gather(codebook, idx_flat):
    info = plsc.get_sparse_core_info()
    nw = info.num_cores * info.num_subcores
    b = idx_flat.shape[0]
    d = codebook.shape[1]
    b_per_w = b // nw
    mesh = plsc.VectorSubcoreMesh(core_axis_name="c", subcore_axis_name="s")

    @functools.partial(
        pl.kernel,
        mesh=mesh,
        out_type=jax.ShapeDtypeStruct((b, d), jnp.float32),
        scratch_types=[
            pltpu.VMEM((b_per_w,), jnp.int32),
            pltpu.VMEM((b_per_w, d), jnp.float32),
            pltpu.SemaphoreType.DMA,
        ],
    )
    def k(table_hbm, idx_hbm, out_hbm, idx_v, rows_v, sem):
        wid = lax.axis_index("s") * info.num_cores + lax.axis_index("c")
        base = wid * b_per_w
        pltpu.sync_copy(idx_hbm.at[pl.ds(base, b_per_w)], idx_v)
        pltpu.async_copy(table_hbm.at[idx_v], rows_v, sem).wait()
        pltpu.sync_copy(rows_v, out_hbm.at[pl.ds(base, b_per_w)])

    return k(codebook, idx_flat)


def kernel(decoder_feat, codebook, positive_indices):
    b, l, c = decoder_feat.shape
    k = codebook.shape[0]
    idx = jnp.clip(positive_indices.reshape(-1), 0, k - 1).astype(jnp.int32)
    pos_emb = _sc_gather(codebook, idx)
    feat2d = decoder_feat.reshape(b * l, c)
    return _tc_loss(feat2d, pos_emb, codebook.T)


# speedup vs baseline: 26.8180x; 26.8180x over previous
import functools

import jax
import jax.numpy as jnp
from jax import lax
from jax.experimental import pallas as pl
from jax.experimental.pallas import tpu as pltpu
from jax.experimental.pallas import tpu_sc as plsc

_TEMP = 0.07
_NHARD = 16
_ROWS_PER_BLOCK = 256
_EPS = 1e-12


def _tc_body(feat_ref, pe_ref, cbt_ref, out_ref):
    f = feat_ref[...]
    pe = pe_ref[...]
    cbt = cbt_ref[...]

    b2 = jnp.sum(cbt * cbt, axis=0, keepdims=True)
    inv_bn = 1.0 / jnp.maximum(jnp.sqrt(b2), _EPS)
    qn = jnp.sqrt(jnp.sum(f * f, axis=1, keepdims=True))
    q = f / jnp.maximum(qn, _EPS)
    a2 = jnp.sum(pe * pe, axis=1, keepdims=True)

    p = jnp.dot(pe, cbt, preferred_element_type=jnp.float32)
    s = jnp.dot(q, cbt, preferred_element_type=jnp.float32) * inv_bn
    dist = jnp.sqrt(jnp.maximum(a2 + b2 - 2.0 * p, 0.0))

    work = dist
    tmin = None
    tk = None
    for i in range(_NHARD + 1):
        m = jnp.min(work, axis=1, keepdims=True)
        if i == 0:
            tmin = m
        tk = m
        if i < _NHARD:
            work = jnp.where(work <= m, jnp.float32(jnp.inf), work)

    neg_mask = jnp.logical_and(dist <= tk, dist > tmin)
    w = jnp.where(neg_mask, jnp.exp(s * (1.0 / _TEMP)), 0.0)
    negsum = jnp.sum(w, axis=1, keepdims=True)

    kpos = pe / jnp.maximum(jnp.sqrt(a2), _EPS)
    pos_l = jnp.sum(q * kpos, axis=1, keepdims=True) * (1.0 / _TEMP)
    row_loss = jnp.log(jnp.exp(pos_l) + negsum) - pos_l
    out_ref[...] = jnp.full((1, 1, 128), jnp.sum(row_loss), jnp.float32)


def _tc_loss(feat2d, pe, cbt):
    bl, c = feat2d.shape
    k = cbt.shape[1]
    r = _ROWS_PER_BLOCK
    nb = bl // r
    partial = pl.pallas_call(
        _tc_body,
        grid=(nb,),
        in_specs=[
            pl.BlockSpec((r, c), lambda i: (i, 0)),
            pl.BlockSpec((r, c), lambda i: (i, 0)),
            pl.BlockSpec((c, k), lambda i: (0, 0)),
        ],
        out_specs=pl.BlockSpec((1, 1, 128), lambda i: (i, 0, 0)),
        out_shape=jax.ShapeDtypeStruct((nb, 1, 128), jnp.float32),
    )(feat2d, pe, cbt)
    return jnp.sum(partial[:, 0, 0]) / bl


def _sc_gather(table, idx_flat):
    info = plsc.get_sparse_core_info()
    nw = info.num_cores * info.num_subcores
    b = idx_flat.shape[0]
    d = table.shape[1]
    b_per_w = b // nw
    mesh = plsc.VectorSubcoreMesh(core_axis_name="c", subcore_axis_name="s")

    @functools.partial(
        pl.kernel,
        mesh=mesh,
        out_type=jax.ShapeDtypeStruct((b, d), jnp.float32),
        scratch_types=[
            pltpu.VMEM((b_per_w,), jnp.int32),
            pltpu.VMEM((b_per_w, d), jnp.float32),
            pltpu.SemaphoreType.DMA,
        ],
    )
    def k(table_hbm, idx_hbm, out_hbm, idx_v, rows_v, sem):
        wid = lax.axis_index("s") * info.num_cores + lax.axis_index("c")
        base = wid * b_per_w
        pltpu.sync_copy(idx_hbm.at[pl.ds(base, b_per_w)], idx_v)
        pltpu.async_copy(table_hbm.at[idx_v], rows_v, sem).wait()
        pltpu.sync_copy(rows_v, out_hbm.at[pl.ds(base, b_per_w)])

    return k(table, idx_flat)


def kernel(decoder_feat, codebook, positive_indices):
    b, l, c = decoder_feat.shape
    k = codebook.shape[0]
    idx = jnp.clip(positive_indices.reshape(-1), 0, k - 1).astype(jnp.int32)
    cb_pad = jnp.pad(codebook, ((0, 0), (0, 128 - c)))
    pos_emb = _sc_gather(cb_pad, idx)[:, :c]
    feat2d = decoder_feat.reshape(b * l, c)
    return _tc_loss(feat2d, pos_emb, codebook.T)


# speedup vs baseline: 30.1211x; 1.1232x over previous
import functools

import jax
import jax.numpy as jnp
from jax import lax
from jax.experimental import pallas as pl
from jax.experimental.pallas import tpu as pltpu
from jax.experimental.pallas import tpu_sc as plsc

_TEMP = 0.07
_NHARD = 16
_ROWS_PER_BLOCK = 256
_EPS = 1e-12


def _tc_body(feat_ref, pe_ref, cbt_ref, out_ref):
    f = feat_ref[...]
    pe = pe_ref[...]
    cbt = cbt_ref[...]
    r = f.shape[0]

    b2 = jnp.sum(cbt * cbt, axis=0, keepdims=True)
    sc_b = (1.0 / _TEMP) / jnp.maximum(jnp.sqrt(b2), _EPS)
    qn = jnp.sqrt(jnp.sum(f * f, axis=1, keepdims=True))
    q = f / jnp.maximum(qn, _EPS)

    a = jnp.concatenate([-2.0 * pe, q], axis=0)
    pq = jnp.dot(a, cbt, preferred_element_type=jnp.float32)
    rank = b2 + pq[:r, :]
    s = pq[r:, :] * sc_b

    work = rank
    tmin = None
    tk = None
    for i in range(_NHARD + 1):
        m = jnp.min(work, axis=1, keepdims=True)
        if i == 0:
            tmin = m
        tk = m
        if i < _NHARD:
            work = jnp.where(work <= m, jnp.float32(jnp.inf), work)

    neg_mask = jnp.logical_and(rank <= tk, rank > tmin)
    w = jnp.where(neg_mask, jnp.exp(s), 0.0)
    negsum = jnp.sum(w, axis=1, keepdims=True)

    a2 = jnp.sum(pe * pe, axis=1, keepdims=True)
    kpos = pe / jnp.maximum(jnp.sqrt(a2), _EPS)
    pos_l = jnp.sum(q * kpos, axis=1, keepdims=True) * (1.0 / _TEMP)
    row_loss = jnp.log(jnp.exp(pos_l) + negsum) - pos_l
    out_ref[...] = jnp.full((1, 1, 128), jnp.sum(row_loss), jnp.float32)


def _tc_loss(feat2d, pe, cbt):
    bl, c = feat2d.shape
    k = cbt.shape[1]
    r = _ROWS_PER_BLOCK
    nb = bl // r
    partial = pl.pallas_call(
        _tc_body,
        grid=(nb,),
        in_specs=[
            pl.BlockSpec((r, c), lambda i: (i, 0)),
            pl.BlockSpec((r, c), lambda i: (i, 0)),
            pl.BlockSpec((c, k), lambda i: (0, 0)),
        ],
        out_specs=pl.BlockSpec((1, 1, 128), lambda i: (i, 0, 0)),
        out_shape=jax.ShapeDtypeStruct((nb, 1, 128), jnp.float32),
    )(feat2d, pe, cbt)
    return jnp.sum(partial[:, 0, 0]) / bl


def _sc_gather(table, idx_flat):
    info = plsc.get_sparse_core_info()
    nw = info.num_cores * info.num_subcores
    b = idx_flat.shape[0]
    d = table.shape[1]
    b_per_w = b // nw
    mesh = plsc.VectorSubcoreMesh(core_axis_name="c", subcore_axis_name="s")

    @functools.partial(
        pl.kernel,
        mesh=mesh,
        out_type=jax.ShapeDtypeStruct((b, d), jnp.float32),
        scratch_types=[
            pltpu.VMEM((b_per_w,), jnp.int32),
            pltpu.VMEM((b_per_w, d), jnp.float32),
            pltpu.SemaphoreType.DMA,
        ],
    )
    def k(table_hbm, idx_hbm, out_hbm, idx_v, rows_v, sem):
        wid = lax.axis_index("s") * info.num_cores + lax.axis_index("c")
        base = wid * b_per_w
        pltpu.sync_copy(idx_hbm.at[pl.ds(base, b_per_w)], idx_v)
        pltpu.async_copy(table_hbm.at[idx_v], rows_v, sem).wait()
        pltpu.sync_copy(rows_v, out_hbm.at[pl.ds(base, b_per_w)])

    return k(table, idx_flat)


def kernel(decoder_feat, codebook, positive_indices):
    b, l, c = decoder_feat.shape
    k = codebook.shape[0]
    idx = jnp.clip(positive_indices.reshape(-1), 0, k - 1).astype(jnp.int32)
    cb_pad = jnp.pad(codebook, ((0, 0), (0, 128 - c)))
    pos_emb = _sc_gather(cb_pad, idx)[:, :c]
    feat2d = decoder_feat.reshape(b * l, c)
    return _tc_loss(feat2d, pos_emb, codebook.T)
